# baseline (speedup 1.0000x reference)
import jax
import jax.numpy as jnp
from jax.experimental import pallas as pl
from jax.experimental.pallas import tpu as pltpu

_M = 4096
_B = 256
_NB = _M // _B
_P = 512
_THRESH = 0.1


def _nms_kernel(payload_ref, bt_ref, scal_ref, res_ref, valid_ref):
    f32 = jnp.float32
    bt = bt_ref[:]
    pre = scal_ref[0, 0]
    post = scal_ref[0, 1]

    cx, cy, cz = bt[0:1], bt[1:2], bt[2:3]
    dx, dy, dz = bt[3:4], bt[4:5], bt[5:6]
    lox, hix = cx - dx / 2.0, cx + dx / 2.0
    loy, hiy = cy - dy / 2.0, cy + dy / 2.0
    loz, hiz = cz - dz / 2.0, cz + dz / 2.0
    volj = dx * dy * dz

    colid = jax.lax.broadcasted_iota(jnp.int32, (1, _M), 1)
    keep_tail = jnp.where(colid < pre, 1.0, 0.0)
    rowid = jax.lax.broadcasted_iota(jnp.int32, (_B, 1), 0)
    tri = jnp.where(colid[:, :_B] > rowid, 1.0, 0.0)

    done_blocks = []
    for ib in range(_NB):
        base = ib * _B
        blk = payload_ref[pl.ds(base, _B), :]
        bcx, bcy, bcz = blk[:, 0:1], blk[:, 1:2], blk[:, 2:3]
        bdx, bdy, bdz = blk[:, 3:4], blk[:, 4:5], blk[:, 5:6]
        blox, bhix = bcx - bdx / 2.0, bcx + bdx / 2.0
        bloy, bhiy = bcy - bdy / 2.0, bcy + bdy / 2.0
        bloz, bhiz = bcz - bdz / 2.0, bcz + bdz / 2.0
        voli = bdx * bdy * bdz

        wx = jnp.maximum(jnp.minimum(bhix, hix[:, base:]) - jnp.maximum(blox, lox[:, base:]), 0.0)
        wy = jnp.maximum(jnp.minimum(bhiy, hiy[:, base:]) - jnp.maximum(bloy, loy[:, base:]), 0.0)
        wz = jnp.maximum(jnp.minimum(bhiz, hiz[:, base:]) - jnp.maximum(bloz, loz[:, base:]), 0.0)
        inter = wx * wy * wz
        union = voli + volj[:, base:] - inter
        iou = inter / jnp.maximum(union, 1e-6)
        tc = jnp.where(iou > _THRESH, 1.0, 0.0)
        tb = tc[:, :_B] * tri

        init = keep_tail[:, :_B]

        def cond(c):
            return jnp.logical_not(c[1])

        def body(c, tb=tb, init=init):
            local, _ = c
            sup = jnp.dot(local, tb, preferred_element_type=f32)
            new = jnp.where(sup > 0.0, 0.0, init)
            return new, jnp.all(new == local)

        local, _ = jax.lax.while_loop(cond, body, (init, jnp.bool_(False)))

        if ib < _NB - 1:
            sup_all = jnp.dot(local, tc[:, _B:], preferred_element_type=f32)
            keep_tail = jnp.where(sup_all > 0.0, 0.0, keep_tail[:, _B:])
        done_blocks.append(local)
    keep = jnp.concatenate(done_blocks, axis=1)

    x = keep
    s = 1
    while s < _M:
        shifted = pltpu.roll(x, s, 1)
        x = x + jnp.where(colid >= s, shifted, 0.0)
        s *= 2
    pos = x - 1.0
    nc = jnp.sum(keep)

    prow = jax.lax.broadcasted_iota(jnp.int32, (_P, 1), 0).astype(f32)
    oh = jnp.where((pos == prow) & (keep > 0.0), 1.0, 0.0)
    res = jnp.dot(oh, payload_ref[:], preferred_element_type=f32,
                  precision=jax.lax.Precision.HIGHEST)
    validc = jnp.where((prow < nc) & (prow < post.astype(f32)), 1.0, 0.0)
    res_ref[:] = res * validc
    valid_ref[:] = validc


def _run(payload, bt, scal):
    return pl.pallas_call(
        _nms_kernel,
        out_shape=[
            jax.ShapeDtypeStruct((_P, 16), jnp.float32),
            jax.ShapeDtypeStruct((_P, 1), jnp.float32),
        ],
        in_specs=[
            pl.BlockSpec(memory_space=pltpu.VMEM),
            pl.BlockSpec(memory_space=pltpu.VMEM),
            pl.BlockSpec(memory_space=pltpu.SMEM),
        ],
        out_specs=[
            pl.BlockSpec(memory_space=pltpu.VMEM),
            pl.BlockSpec(memory_space=pltpu.VMEM),
        ],
    )(payload, bt, scal)


def kernel(boxes, scores, pre_maxsize, post_max_size):
    f32 = jnp.float32
    s_sorted, order = jax.lax.top_k(scores, _M)
    b = boxes[order]
    b8 = jnp.pad(b, ((0, 0), (0, 1)))
    payload = jnp.concatenate(
        [b8, order.astype(f32)[:, None], s_sorted[:, None],
         jnp.zeros((_M, 6), f32)], axis=1)
    bt = jnp.transpose(b8)
    scal = jnp.stack([pre_maxsize, post_max_size]).astype(jnp.int32).reshape(1, 2)
    res, validf = _run(payload, bt, scal)
    selected_boxes = res[:500, :7]
    sel_global = res[:500, 8].astype(jnp.int32)
    selected_scores = res[:500, 9]
    valid = validf[:500, 0] > 0.5
    return selected_boxes, selected_scores, sel_global, valid


# speedup vs baseline: 1.0410x; 1.0410x over previous
import jax
import jax.numpy as jnp
from jax.experimental import pallas as pl
from jax.experimental.pallas import tpu as pltpu

_M = 4096
_B = 256
_NB = _M // _B
_P = 512
_THRESH = 0.1


def _nms_kernel(payload_ref, bt_ref, scal_ref, res_ref, valid_ref):
    f32 = jnp.float32
    bt = bt_ref[:]
    pre = scal_ref[0, 0]
    post = scal_ref[0, 1]

    cx, cy, cz = bt[0:1], bt[1:2], bt[2:3]
    dx, dy, dz = bt[3:4], bt[4:5], bt[5:6]
    lox, hix = cx - dx / 2.0, cx + dx / 2.0
    loy, hiy = cy - dy / 2.0, cy + dy / 2.0
    loz, hiz = cz - dz / 2.0, cz + dz / 2.0
    volj = dx * dy * dz

    colid = jax.lax.broadcasted_iota(jnp.int32, (1, _M), 1)
    keep_tail = jnp.where(colid < pre, 1.0, 0.0)
    rowid = jax.lax.broadcasted_iota(jnp.int32, (_B, 1), 0)
    tri = jnp.where(colid[:, :_B] > rowid, 1.0, 0.0)

    done_blocks = []
    for ib in range(_NB):
        base = ib * _B
        blk = payload_ref[pl.ds(base, _B), :]
        bcx, bcy, bcz = blk[:, 0:1], blk[:, 1:2], blk[:, 2:3]
        bdx, bdy, bdz = blk[:, 3:4], blk[:, 4:5], blk[:, 5:6]
        blox, bhix = bcx - bdx / 2.0, bcx + bdx / 2.0
        bloy, bhiy = bcy - bdy / 2.0, bcy + bdy / 2.0
        bloz, bhiz = bcz - bdz / 2.0, bcz + bdz / 2.0
        voli = bdx * bdy * bdz

        wx = jnp.maximum(jnp.minimum(bhix, hix[:, base:]) - jnp.maximum(blox, lox[:, base:]), 0.0)
        wy = jnp.maximum(jnp.minimum(bhiy, hiy[:, base:]) - jnp.maximum(bloy, loy[:, base:]), 0.0)
        wz = jnp.maximum(jnp.minimum(bhiz, hiz[:, base:]) - jnp.maximum(bloz, loz[:, base:]), 0.0)
        inter = wx * wy * wz
        union = voli + volj[:, base:] - inter
        iou = inter / jnp.maximum(union, 1e-6)
        tc = jnp.where(iou > _THRESH, 1.0, 0.0)
        tb = tc[:, :_B] * tri

        init = keep_tail[:, :_B]

        def cond(c):
            return jnp.logical_not(c[1])

        def body(c, tb=tb, init=init):
            local, _ = c
            sup = jnp.dot(local, tb, preferred_element_type=f32)
            new = jnp.where(sup > 0.0, 0.0, init)
            return new, jnp.all(new == local)

        local, _ = jax.lax.while_loop(cond, body, (init, jnp.bool_(False)))

        if ib < _NB - 1:
            sup_all = jnp.dot(local, tc[:, _B:], preferred_element_type=f32)
            keep_tail = jnp.where(sup_all > 0.0, 0.0, keep_tail[:, _B:])
        done_blocks.append(local)
    keep = jnp.concatenate(done_blocks, axis=1)

    keep2 = keep.reshape(32, 128)
    u_r = jax.lax.broadcasted_iota(jnp.int32, (128, 128), 0)
    u_c = jax.lax.broadcasted_iota(jnp.int32, (128, 128), 1)
    umat = jnp.where(u_r <= u_c, 1.0, 0.0)
    incl = jnp.dot(keep2, umat, preferred_element_type=f32)
    rowtot = incl[:, 127:128]
    s_r = jax.lax.broadcasted_iota(jnp.int32, (32, 32), 0)
    s_c = jax.lax.broadcasted_iota(jnp.int32, (32, 32), 1)
    smat = jnp.where(s_c < s_r, 1.0, 0.0)
    offs = jnp.dot(smat, rowtot, preferred_element_type=f32)
    pos = (incl + offs - 1.0).reshape(1, _M)
    nc = jnp.sum(keep)

    prow = jax.lax.broadcasted_iota(jnp.int32, (_P, 1), 0).astype(f32)
    oh = jnp.where((pos == prow) & (keep > 0.0), 1.0, 0.0)
    payload = payload_ref[:]
    l1 = payload.astype(jnp.bfloat16).astype(f32)
    r1 = payload - l1
    l2 = r1.astype(jnp.bfloat16).astype(f32)
    l3 = r1 - l2
    res = (jnp.dot(oh, l1, preferred_element_type=f32)
           + jnp.dot(oh, l2, preferred_element_type=f32)
           + jnp.dot(oh, l3, preferred_element_type=f32))
    validc = jnp.where((prow < nc) & (prow < post.astype(f32)), 1.0, 0.0)
    res_ref[:] = res * validc
    valid_ref[:] = validc


def _run(payload, bt, scal):
    return pl.pallas_call(
        _nms_kernel,
        out_shape=[
            jax.ShapeDtypeStruct((_P, 16), jnp.float32),
            jax.ShapeDtypeStruct((_P, 1), jnp.float32),
        ],
        in_specs=[
            pl.BlockSpec(memory_space=pltpu.VMEM),
            pl.BlockSpec(memory_space=pltpu.VMEM),
            pl.BlockSpec(memory_space=pltpu.SMEM),
        ],
        out_specs=[
            pl.BlockSpec(memory_space=pltpu.VMEM),
            pl.BlockSpec(memory_space=pltpu.VMEM),
        ],
    )(payload, bt, scal)


def kernel(boxes, scores, pre_maxsize, post_max_size):
    f32 = jnp.float32
    s_sorted, order = jax.lax.top_k(scores, _M)
    b = boxes[order]
    b8 = jnp.pad(b, ((0, 0), (0, 1)))
    payload = jnp.concatenate(
        [b8, order.astype(f32)[:, None], s_sorted[:, None],
         jnp.zeros((_M, 6), f32)], axis=1)
    bt = jnp.transpose(b8)
    scal = jnp.stack([pre_maxsize, post_max_size]).astype(jnp.int32).reshape(1, 2)
    res, validf = _run(payload, bt, scal)
    selected_boxes = res[:500, :7]
    sel_global = res[:500, 8].astype(jnp.int32)
    selected_scores = res[:500, 9]
    valid = validf[:500, 0] > 0.5
    return selected_boxes, selected_scores, sel_global, valid
